# baseline (speedup 1.0000x reference)
import jax
import jax.numpy as jnp
from jax.experimental import pallas as pl

N = 2048
F_IN = 256
H1 = 8
D1 = 32
F_HID = 256
D2 = 32
BI = 256
SLOPE = 0.2
LOG2E = 1.4426950408889634


def _proj1_kernel(x_ref, w_ref, al_ref, ar_ref, g_ref, el_ref, ert_ref):
    g = jnp.dot(x_ref[...], w_ref[...], preferred_element_type=jnp.float32)
    g_ref[...] = g
    el_ref[...] = jnp.dot(g, al_ref[...], preferred_element_type=jnp.float32) * LOG2E
    er = jnp.dot(g, ar_ref[...], preferred_element_type=jnp.float32) * LOG2E
    ert_ref[...] = er.T


def _attend(el_col, er_row, adjf, vaug, vmean):
    ermax = jnp.max(er_row, axis=1, keepdims=True)
    mrow = el_col + ermax
    m = jnp.maximum(mrow, SLOPE * mrow)
    el_m = el_col - m
    c = (SLOPE - 1.0) * m
    u = el_m + er_row
    t = SLOPE * u + c
    p = jnp.exp2(jnp.maximum(u, t))
    pm = p * adjf
    od = jnp.dot(pm, vaug, preferred_element_type=jnp.float32)
    o, denom = od[:, :-1], od[:, -1:]
    safe = denom > 0.0
    r = 1.0 / jnp.where(safe, denom, 1.0)
    return jnp.where(safe, o * r, vmean)


def _layer1_kernel(g_ref, el_ref, ert_ref, adj_ref, w2_ref, a2l_ref, a2r_ref,
                   g2_ref, el2_ref, er2_ref):
    adjf = adj_ref[...].astype(jnp.float32)
    g = g_ref[...]
    el = el_ref[...]
    ones = jnp.ones((N, 1), dtype=jnp.float32)
    outs = []
    for h in range(H1):
        v = g[:, h * D1:(h + 1) * D1]
        vaug = jnp.concatenate([v, ones], axis=1)
        vmean = jnp.sum(v, axis=0, keepdims=True) * (1.0 / N)
        outs.append(_attend(el[:, h:h + 1], ert_ref[h:h + 1, :], adjf, vaug, vmean))
    h1 = jnp.concatenate(outs, axis=1)
    h1 = jnp.where(h1 > 0, h1, jnp.exp(jnp.minimum(h1, 0.0)) - 1.0)
    g2 = jnp.dot(h1, w2_ref[...], preferred_element_type=jnp.float32)
    g2_ref[...] = g2
    el2_ref[...] = jnp.dot(g2, a2l_ref[...], preferred_element_type=jnp.float32) * LOG2E
    er2_ref[...] = jnp.dot(g2, a2r_ref[...], preferred_element_type=jnp.float32) * LOG2E


def _layer2_kernel(g2_ref, el2_ref, er2t_ref, adj_ref, out_ref):
    g2 = g2_ref[...]
    vaug = jnp.concatenate([g2, jnp.ones((N, 1), dtype=jnp.float32)], axis=1)
    vmean = jnp.sum(g2, axis=0, keepdims=True) * (1.0 / N)
    adjf = adj_ref[...].astype(jnp.float32)
    out_ref[...] = _attend(el2_ref[...], er2t_ref[...], adjf, vaug, vmean)


def kernel(x, adj_mat, W1, a1_l, a1_r, W2, a2_l, a2_r):
    adj = adj_mat.reshape(N, N)
    eye = jnp.eye(H1, dtype=jnp.float32)
    Al = (eye[:, None, :] * a1_l[None, :, None]).reshape(F_HID, H1)
    Ar = (eye[:, None, :] * a1_r[None, :, None]).reshape(F_HID, H1)
    a2l = a2_l.reshape(D2, 1)
    a2r = a2_r.reshape(D2, 1)

    g1, el1, er1t = pl.pallas_call(
        _proj1_kernel,
        out_shape=[
            jax.ShapeDtypeStruct((N, F_HID), jnp.float32),
            jax.ShapeDtypeStruct((N, H1), jnp.float32),
            jax.ShapeDtypeStruct((H1, N), jnp.float32),
        ],
    )(x, W1, Al, Ar)

    nblk = N // BI
    g2, el2, er2 = pl.pallas_call(
        _layer1_kernel,
        grid=(nblk,),
        in_specs=[
            pl.BlockSpec((N, F_HID), lambda i: (0, 0)),
            pl.BlockSpec((BI, H1), lambda i: (i, 0)),
            pl.BlockSpec((H1, N), lambda i: (0, 0)),
            pl.BlockSpec((BI, N), lambda i: (i, 0)),
            pl.BlockSpec((F_HID, D2), lambda i: (0, 0)),
            pl.BlockSpec((D2, 1), lambda i: (0, 0)),
            pl.BlockSpec((D2, 1), lambda i: (0, 0)),
        ],
        out_specs=[
            pl.BlockSpec((BI, D2), lambda i: (i, 0)),
            pl.BlockSpec((BI, 1), lambda i: (i, 0)),
            pl.BlockSpec((BI, 1), lambda i: (i, 0)),
        ],
        out_shape=[
            jax.ShapeDtypeStruct((N, D2), jnp.float32),
            jax.ShapeDtypeStruct((N, 1), jnp.float32),
            jax.ShapeDtypeStruct((N, 1), jnp.float32),
        ],
    )(g1, el1, er1t, adj, W2, a2l, a2r)

    er2t = er2.reshape(1, N)
    out = pl.pallas_call(
        _layer2_kernel,
        grid=(nblk,),
        in_specs=[
            pl.BlockSpec((N, D2), lambda i: (0, 0)),
            pl.BlockSpec((BI, 1), lambda i: (i, 0)),
            pl.BlockSpec((1, N), lambda i: (0, 0)),
            pl.BlockSpec((BI, N), lambda i: (i, 0)),
        ],
        out_specs=pl.BlockSpec((BI, D2), lambda i: (i, 0)),
        out_shape=jax.ShapeDtypeStruct((N, D2), jnp.float32),
    )(g2, el2, er2t, adj)
    return out


# speedup vs baseline: 1.1333x; 1.1333x over previous
import jax
import jax.numpy as jnp
from jax.experimental import pallas as pl

N = 2048
F_IN = 256
H1 = 8
D1 = 32
F_HID = 256
D2 = 32
BI = 512
SLOPE = 0.2
LOG2E = 1.4426950408889634


def _proj1_kernel(x_ref, w_ref, al_ref, ar_ref, g_ref, el_ref, ert_ref):
    g = jnp.dot(x_ref[...], w_ref[...], preferred_element_type=jnp.float32)
    g_ref[...] = g
    el_ref[...] = jnp.dot(g, al_ref[...], preferred_element_type=jnp.float32) * LOG2E
    er = jnp.dot(g, ar_ref[...], preferred_element_type=jnp.float32) * LOG2E
    ert_ref[...] = er.T


def _attend(el_col, er_row, adjf, vaug, vmean):
    ermax = jnp.max(er_row, axis=1, keepdims=True)
    mrow = el_col + ermax
    m = jnp.maximum(mrow, SLOPE * mrow)
    s = el_col + er_row
    l = jnp.maximum(s, SLOPE * s)
    pm = jnp.where(adjf, jnp.exp2(l - m), 0.0)
    od = jnp.dot(pm, vaug, preferred_element_type=jnp.float32)
    o, denom = od[:, :-1], od[:, -1:]
    safe = denom > 0.0
    r = 1.0 / jnp.where(safe, denom, 1.0)
    return jnp.where(safe, o * r, vmean)


def _layer1_kernel(g_ref, el_ref, ert_ref, adj_ref, w2_ref, a2l_ref, a2r_ref,
                   g2_ref, el2_ref, er2_ref):
    adjf = adj_ref[...]
    g = g_ref[...]
    el = el_ref[...]
    ones = jnp.ones((N, 1), dtype=jnp.float32)
    outs = []
    for h in range(H1):
        v = g[:, h * D1:(h + 1) * D1]
        vaug = jnp.concatenate([v, ones], axis=1)
        vmean = jnp.sum(v, axis=0, keepdims=True) * (1.0 / N)
        outs.append(_attend(el[:, h:h + 1], ert_ref[h:h + 1, :], adjf, vaug, vmean))
    h1 = jnp.concatenate(outs, axis=1)
    h1 = jnp.where(h1 > 0, h1, jnp.exp(jnp.minimum(h1, 0.0)) - 1.0)
    g2 = jnp.dot(h1, w2_ref[...], preferred_element_type=jnp.float32)
    g2_ref[...] = g2
    el2_ref[...] = jnp.dot(g2, a2l_ref[...], preferred_element_type=jnp.float32) * LOG2E
    er2_ref[...] = jnp.dot(g2, a2r_ref[...], preferred_element_type=jnp.float32) * LOG2E


def _layer2_kernel(g2_ref, el2_ref, er2t_ref, adj_ref, out_ref):
    g2 = g2_ref[...]
    vaug = jnp.concatenate([g2, jnp.ones((N, 1), dtype=jnp.float32)], axis=1)
    vmean = jnp.sum(g2, axis=0, keepdims=True) * (1.0 / N)
    out_ref[...] = _attend(el2_ref[...], er2t_ref[...], adj_ref[...], vaug, vmean)


def kernel(x, adj_mat, W1, a1_l, a1_r, W2, a2_l, a2_r):
    adj = adj_mat.reshape(N, N)
    eye = jnp.eye(H1, dtype=jnp.float32)
    Al = (eye[:, None, :] * a1_l[None, :, None]).reshape(F_HID, H1)
    Ar = (eye[:, None, :] * a1_r[None, :, None]).reshape(F_HID, H1)
    a2l = a2_l.reshape(D2, 1)
    a2r = a2_r.reshape(D2, 1)

    g1, el1, er1t = pl.pallas_call(
        _proj1_kernel,
        out_shape=[
            jax.ShapeDtypeStruct((N, F_HID), jnp.float32),
            jax.ShapeDtypeStruct((N, H1), jnp.float32),
            jax.ShapeDtypeStruct((H1, N), jnp.float32),
        ],
    )(x, W1, Al, Ar)

    nblk = N // BI
    g2, el2, er2 = pl.pallas_call(
        _layer1_kernel,
        grid=(nblk,),
        in_specs=[
            pl.BlockSpec((N, F_HID), lambda i: (0, 0)),
            pl.BlockSpec((BI, H1), lambda i: (i, 0)),
            pl.BlockSpec((H1, N), lambda i: (0, 0)),
            pl.BlockSpec((BI, N), lambda i: (i, 0)),
            pl.BlockSpec((F_HID, D2), lambda i: (0, 0)),
            pl.BlockSpec((D2, 1), lambda i: (0, 0)),
            pl.BlockSpec((D2, 1), lambda i: (0, 0)),
        ],
        out_specs=[
            pl.BlockSpec((BI, D2), lambda i: (i, 0)),
            pl.BlockSpec((BI, 1), lambda i: (i, 0)),
            pl.BlockSpec((BI, 1), lambda i: (i, 0)),
        ],
        out_shape=[
            jax.ShapeDtypeStruct((N, D2), jnp.float32),
            jax.ShapeDtypeStruct((N, 1), jnp.float32),
            jax.ShapeDtypeStruct((N, 1), jnp.float32),
        ],
    )(g1, el1, er1t, adj, W2, a2l, a2r)

    er2t = er2.reshape(1, N)
    out = pl.pallas_call(
        _layer2_kernel,
        grid=(nblk,),
        in_specs=[
            pl.BlockSpec((N, D2), lambda i: (0, 0)),
            pl.BlockSpec((BI, 1), lambda i: (i, 0)),
            pl.BlockSpec((1, N), lambda i: (0, 0)),
            pl.BlockSpec((BI, N), lambda i: (i, 0)),
        ],
        out_specs=pl.BlockSpec((BI, D2), lambda i: (i, 0)),
        out_shape=jax.ShapeDtypeStruct((N, D2), jnp.float32),
    )(g2, el2, er2t, adj)
    return out


# speedup vs baseline: 1.1700x; 1.0324x over previous
import jax
import jax.numpy as jnp
from jax.experimental import pallas as pl

N = 2048
F_IN = 256
H1 = 8
D1 = 32
F_HID = 256
D2 = 32
BI = 1024
SLOPE = 0.2
LOG2E = 1.4426950408889634


def _proj1_kernel(x_ref, w_ref, al_ref, ar_ref, g_ref, el_ref, ert_ref):
    g = jnp.dot(x_ref[...], w_ref[...], preferred_element_type=jnp.float32)
    g_ref[...] = g
    el_ref[...] = jnp.dot(g, al_ref[...], preferred_element_type=jnp.float32) * LOG2E
    er = jnp.dot(g, ar_ref[...], preferred_element_type=jnp.float32) * LOG2E
    ert_ref[...] = er.T


def _attend(el_col, er_row, adjf, vaug, vmean):
    ermax = jnp.max(er_row, axis=1, keepdims=True)
    mrow = el_col + ermax
    m = jnp.maximum(mrow, SLOPE * mrow)
    s = el_col + er_row
    l = jnp.maximum(s, SLOPE * s)
    pm = jnp.where(adjf, jnp.exp2(l - m), 0.0)
    od = jnp.dot(pm, vaug, preferred_element_type=jnp.float32)
    o, denom = od[:, :-1], od[:, -1:]
    safe = denom > 0.0
    r = 1.0 / jnp.where(safe, denom, 1.0)
    return jnp.where(safe, o * r, vmean)


def _layer1_kernel(g_ref, el_ref, ert_ref, adj_ref, w2_ref, a2l_ref, a2r_ref,
                   g2_ref, el2_ref, er2_ref):
    adjf = adj_ref[...]
    g = g_ref[...]
    el = el_ref[...]
    ones = jnp.ones((N, 1), dtype=jnp.float32)
    outs = []
    for h in range(H1):
        v = g[:, h * D1:(h + 1) * D1]
        vaug = jnp.concatenate([v, ones], axis=1)
        vmean = jnp.sum(v, axis=0, keepdims=True) * (1.0 / N)
        outs.append(_attend(el[:, h:h + 1], ert_ref[h:h + 1, :], adjf, vaug, vmean))
    h1 = jnp.concatenate(outs, axis=1)
    h1 = jnp.where(h1 > 0, h1, jnp.exp(jnp.minimum(h1, 0.0)) - 1.0)
    g2 = jnp.dot(h1, w2_ref[...], preferred_element_type=jnp.float32)
    g2_ref[...] = g2
    el2_ref[...] = jnp.dot(g2, a2l_ref[...], preferred_element_type=jnp.float32) * LOG2E
    er2_ref[...] = jnp.dot(g2, a2r_ref[...], preferred_element_type=jnp.float32) * LOG2E


def _layer2_kernel(g2_ref, el2_ref, er2t_ref, adj_ref, out_ref):
    g2 = g2_ref[...]
    vaug = jnp.concatenate([g2, jnp.ones((N, 1), dtype=jnp.float32)], axis=1)
    vmean = jnp.sum(g2, axis=0, keepdims=True) * (1.0 / N)
    out_ref[...] = _attend(el2_ref[...], er2t_ref[...], adj_ref[...], vaug, vmean)


def kernel(x, adj_mat, W1, a1_l, a1_r, W2, a2_l, a2_r):
    adj = adj_mat.reshape(N, N)
    eye = jnp.eye(H1, dtype=jnp.float32)
    Al = (eye[:, None, :] * a1_l[None, :, None]).reshape(F_HID, H1)
    Ar = (eye[:, None, :] * a1_r[None, :, None]).reshape(F_HID, H1)
    a2l = a2_l.reshape(D2, 1)
    a2r = a2_r.reshape(D2, 1)

    g1, el1, er1t = pl.pallas_call(
        _proj1_kernel,
        out_shape=[
            jax.ShapeDtypeStruct((N, F_HID), jnp.float32),
            jax.ShapeDtypeStruct((N, H1), jnp.float32),
            jax.ShapeDtypeStruct((H1, N), jnp.float32),
        ],
    )(x, W1, Al, Ar)

    nblk = N // BI
    g2, el2, er2 = pl.pallas_call(
        _layer1_kernel,
        grid=(nblk,),
        in_specs=[
            pl.BlockSpec((N, F_HID), lambda i: (0, 0)),
            pl.BlockSpec((BI, H1), lambda i: (i, 0)),
            pl.BlockSpec((H1, N), lambda i: (0, 0)),
            pl.BlockSpec((BI, N), lambda i: (i, 0)),
            pl.BlockSpec((F_HID, D2), lambda i: (0, 0)),
            pl.BlockSpec((D2, 1), lambda i: (0, 0)),
            pl.BlockSpec((D2, 1), lambda i: (0, 0)),
        ],
        out_specs=[
            pl.BlockSpec((BI, D2), lambda i: (i, 0)),
            pl.BlockSpec((BI, 1), lambda i: (i, 0)),
            pl.BlockSpec((BI, 1), lambda i: (i, 0)),
        ],
        out_shape=[
            jax.ShapeDtypeStruct((N, D2), jnp.float32),
            jax.ShapeDtypeStruct((N, 1), jnp.float32),
            jax.ShapeDtypeStruct((N, 1), jnp.float32),
        ],
    )(g1, el1, er1t, adj, W2, a2l, a2r)

    er2t = er2.reshape(1, N)
    out = pl.pallas_call(
        _layer2_kernel,
        grid=(nblk,),
        in_specs=[
            pl.BlockSpec((N, D2), lambda i: (0, 0)),
            pl.BlockSpec((BI, 1), lambda i: (i, 0)),
            pl.BlockSpec((1, N), lambda i: (0, 0)),
            pl.BlockSpec((BI, N), lambda i: (i, 0)),
        ],
        out_specs=pl.BlockSpec((BI, D2), lambda i: (i, 0)),
        out_shape=jax.ShapeDtypeStruct((N, D2), jnp.float32),
    )(g2, el2, er2t, adj)
    return out


# speedup vs baseline: 1.3146x; 1.1236x over previous
import jax
import jax.numpy as jnp
from jax.experimental import pallas as pl
from jax.experimental.pallas import tpu as pltpu

N = 2048
F_IN = 256
H1 = 8
D1 = 32
F_HID = 256
D2 = 32
BI = 1024
NBLK = N // BI
SLOPE = 0.2
LOG2E = 1.4426950408889634


def _attend(el_col, er_row, adj, vaug, vmean):
    ermax = jnp.max(er_row, axis=1, keepdims=True)
    mrow = el_col + ermax
    m = jnp.maximum(mrow, SLOPE * mrow)
    s = el_col + er_row
    l = jnp.maximum(s, SLOPE * s)
    pm = jnp.where(adj, jnp.exp2(l - m), 0.0)
    od = jnp.dot(pm, vaug, preferred_element_type=jnp.float32)
    o, denom = od[:, :-1], od[:, -1:]
    safe = denom > 0.0
    r = 1.0 / jnp.where(safe, denom, 1.0)
    return jnp.where(safe, o * r, vmean)


def _fused_kernel(x_ref, w1_ref, al_ref, ar_ref, adj_ref, w2_ref, a2l_ref,
                  a2r_ref, out_ref, g1_s, el1_s, ert_s, g2_s, el2_s, er2_s,
                  er2t_s):
    i = pl.program_id(0)

    @pl.when(i == 0)
    def _prologue():
        g = jnp.dot(x_ref[...], w1_ref[...], preferred_element_type=jnp.float32)
        g1_s[...] = g
        el1_s[...] = jnp.dot(g, al_ref[...], preferred_element_type=jnp.float32) * LOG2E
        er = jnp.dot(g, ar_ref[...], preferred_element_type=jnp.float32) * LOG2E
        ert_s[...] = er.T

    @pl.when(i < NBLK)
    def _layer1():
        row0 = i * BI
        adj = adj_ref[...]
        g = g1_s[...]
        el = el1_s[pl.ds(row0, BI), :]
        ones = jnp.ones((N, 1), dtype=jnp.float32)
        outs = []
        for h in range(H1):
            v = g[:, h * D1:(h + 1) * D1]
            vaug = jnp.concatenate([v, ones], axis=1)
            vmean = jnp.sum(v, axis=0, keepdims=True) * (1.0 / N)
            outs.append(_attend(el[:, h:h + 1], ert_s[h:h + 1, :], adj, vaug, vmean))
        h1 = jnp.concatenate(outs, axis=1)
        h1 = jnp.where(h1 > 0, h1, jnp.exp(jnp.minimum(h1, 0.0)) - 1.0)
        g2 = jnp.dot(h1, w2_ref[...], preferred_element_type=jnp.float32)
        g2_s[pl.ds(row0, BI), :] = g2
        el2_s[pl.ds(row0, BI), :] = jnp.dot(
            g2, a2l_ref[...], preferred_element_type=jnp.float32) * LOG2E
        er2_s[pl.ds(row0, BI), :] = jnp.dot(
            g2, a2r_ref[...], preferred_element_type=jnp.float32) * LOG2E

    @pl.when(i == NBLK)
    def _transpose_er2():
        er2t_s[...] = er2_s[...].T

    @pl.when(i >= NBLK)
    def _layer2():
        row0 = (i - NBLK) * BI
        g2 = g2_s[...]
        vaug = jnp.concatenate([g2, jnp.ones((N, 1), dtype=jnp.float32)], axis=1)
        vmean = jnp.sum(g2, axis=0, keepdims=True) * (1.0 / N)
        out_ref[...] = _attend(el2_s[pl.ds(row0, BI), :], er2t_s[...],
                               adj_ref[...], vaug, vmean)


def kernel(x, adj_mat, W1, a1_l, a1_r, W2, a2_l, a2_r):
    adj = adj_mat.reshape(N, N)
    eye = jnp.eye(H1, dtype=jnp.float32)
    Al = (eye[:, None, :] * a1_l[None, :, None]).reshape(F_HID, H1)
    Ar = (eye[:, None, :] * a1_r[None, :, None]).reshape(F_HID, H1)
    a2l = a2_l.reshape(D2, 1)
    a2r = a2_r.reshape(D2, 1)

    out = pl.pallas_call(
        _fused_kernel,
        grid=(2 * NBLK,),
        in_specs=[
            pl.BlockSpec((N, F_IN), lambda i: (0, 0)),
            pl.BlockSpec((F_IN, F_HID), lambda i: (0, 0)),
            pl.BlockSpec((F_HID, H1), lambda i: (0, 0)),
            pl.BlockSpec((F_HID, H1), lambda i: (0, 0)),
            pl.BlockSpec((BI, N), lambda i: (jax.lax.rem(i, NBLK), 0)),
            pl.BlockSpec((F_HID, D2), lambda i: (0, 0)),
            pl.BlockSpec((D2, 1), lambda i: (0, 0)),
            pl.BlockSpec((D2, 1), lambda i: (0, 0)),
        ],
        out_specs=pl.BlockSpec((BI, D2), lambda i: (jnp.maximum(i - NBLK, 0), 0)),
        out_shape=jax.ShapeDtypeStruct((N, D2), jnp.float32),
        scratch_shapes=[
            pltpu.VMEM((N, F_HID), jnp.float32),
            pltpu.VMEM((N, H1), jnp.float32),
            pltpu.VMEM((H1, N), jnp.float32),
            pltpu.VMEM((N, D2), jnp.float32),
            pltpu.VMEM((N, 1), jnp.float32),
            pltpu.VMEM((N, 1), jnp.float32),
            pltpu.VMEM((1, N), jnp.float32),
        ],
    )(x, W1, Al, Ar, adj, W2, a2l, a2r)
    return out


# speedup vs baseline: 1.3648x; 1.0382x over previous
import jax
import jax.numpy as jnp
from jax.experimental import pallas as pl
from jax.experimental.pallas import tpu as pltpu

N = 2048
F_IN = 256
H1 = 8
D1 = 32
F_HID = 256
D2 = 32
BI = 1024
NBLK = N // BI
SLOPE = 0.2
LOG2E = 1.4426950408889634


def _attend(el_col, er_row, adj, vaug, vmean):
    ermax = jnp.max(er_row, axis=1, keepdims=True)
    mrow = el_col + ermax
    m = jnp.maximum(mrow, SLOPE * mrow)
    s = el_col + er_row
    l = jnp.maximum(s, SLOPE * s)
    pm = jnp.exp2(l - m) * adj
    od = jnp.dot(pm, vaug, preferred_element_type=jnp.float32)
    o, denom = od[:, :-1], od[:, -1:]
    safe = denom > 0.0
    r = 1.0 / jnp.where(safe, denom, 1.0)
    return jnp.where(safe, o * r, vmean)


def _fused_kernel(x_ref, w1_ref, al_ref, ar_ref, adj_ref, w2_ref, a2l_ref,
                  a2r_ref, out_ref, g1_s, el1_s, ert_s, g2_s, el2_s, er2_s,
                  er2t_s):
    i = pl.program_id(0)

    @pl.when(i == 0)
    def _prologue():
        g = jnp.dot(x_ref[...], w1_ref[...], preferred_element_type=jnp.float32)
        g1_s[...] = g
        el1_s[...] = jnp.dot(g, al_ref[...], preferred_element_type=jnp.float32) * LOG2E
        er = jnp.dot(g, ar_ref[...], preferred_element_type=jnp.float32) * LOG2E
        ert_s[...] = er.T

    @pl.when(i < NBLK)
    def _layer1():
        row0 = i * BI
        adj = adj_ref[...]
        g = g1_s[...]
        el = el1_s[pl.ds(row0, BI), :]
        ones = jnp.ones((N, 1), dtype=jnp.float32)
        outs = []
        for h in range(H1):
            v = g[:, h * D1:(h + 1) * D1]
            vaug = jnp.concatenate([v, ones], axis=1)
            vmean = jnp.sum(v, axis=0, keepdims=True) * (1.0 / N)
            outs.append(_attend(el[:, h:h + 1], ert_s[h:h + 1, :], adj, vaug, vmean))
        h1 = jnp.concatenate(outs, axis=1)
        h1 = jnp.where(h1 > 0, h1, jnp.exp(jnp.minimum(h1, 0.0)) - 1.0)
        g2 = jnp.dot(h1, w2_ref[...], preferred_element_type=jnp.float32)
        g2_s[pl.ds(row0, BI), :] = g2
        el2_s[pl.ds(row0, BI), :] = jnp.dot(
            g2, a2l_ref[...], preferred_element_type=jnp.float32) * LOG2E
        er2_s[pl.ds(row0, BI), :] = jnp.dot(
            g2, a2r_ref[...], preferred_element_type=jnp.float32) * LOG2E

    @pl.when(i == NBLK)
    def _transpose_er2():
        er2t_s[...] = er2_s[...].T

    @pl.when(i >= NBLK)
    def _layer2():
        row0 = (i - NBLK) * BI
        g2 = g2_s[...]
        vaug = jnp.concatenate([g2, jnp.ones((N, 1), dtype=jnp.float32)], axis=1)
        vmean = jnp.sum(g2, axis=0, keepdims=True) * (1.0 / N)
        out_ref[...] = _attend(el2_s[pl.ds(row0, BI), :], er2t_s[...],
                               adj_ref[...], vaug, vmean)


def kernel(x, adj_mat, W1, a1_l, a1_r, W2, a2_l, a2_r):
    adj = adj_mat.reshape(N, N).astype(jnp.float32)
    eye = jnp.eye(H1, dtype=jnp.float32)
    Al = (eye[:, None, :] * a1_l[None, :, None]).reshape(F_HID, H1)
    Ar = (eye[:, None, :] * a1_r[None, :, None]).reshape(F_HID, H1)
    a2l = a2_l.reshape(D2, 1)
    a2r = a2_r.reshape(D2, 1)

    out = pl.pallas_call(
        _fused_kernel,
        grid=(2 * NBLK,),
        in_specs=[
            pl.BlockSpec((N, F_IN), lambda i: (0, 0)),
            pl.BlockSpec((F_IN, F_HID), lambda i: (0, 0)),
            pl.BlockSpec((F_HID, H1), lambda i: (0, 0)),
            pl.BlockSpec((F_HID, H1), lambda i: (0, 0)),
            pl.BlockSpec((BI, N), lambda i: (jax.lax.rem(i, NBLK), 0)),
            pl.BlockSpec((F_HID, D2), lambda i: (0, 0)),
            pl.BlockSpec((D2, 1), lambda i: (0, 0)),
            pl.BlockSpec((D2, 1), lambda i: (0, 0)),
        ],
        out_specs=pl.BlockSpec((BI, D2), lambda i: (jnp.maximum(i - NBLK, 0), 0)),
        out_shape=jax.ShapeDtypeStruct((N, D2), jnp.float32),
        scratch_shapes=[
            pltpu.VMEM((N, F_HID), jnp.float32),
            pltpu.VMEM((N, H1), jnp.float32),
            pltpu.VMEM((H1, N), jnp.float32),
            pltpu.VMEM((N, D2), jnp.float32),
            pltpu.VMEM((N, 1), jnp.float32),
            pltpu.VMEM((N, 1), jnp.float32),
            pltpu.VMEM((1, N), jnp.float32),
        ],
    )(x, W1, Al, Ar, adj, W2, a2l, a2r)
    return out


# speedup vs baseline: 1.6798x; 1.2308x over previous
import jax
import jax.numpy as jnp
from jax.experimental import pallas as pl
from jax.experimental.pallas import tpu as pltpu

N = 2048
F_IN = 256
H1 = 8
D1 = 32
F_HID = 256
D2 = 32
BI = 1024
NBLK = N // BI
SLOPE = 0.2
LOG2E = 1.4426950408889634
BF = jnp.bfloat16


def _attend(el_col, er_row, adj, vaug, vmean):
    ermax = jnp.max(er_row.astype(jnp.float32), axis=1, keepdims=True)
    mrow = el_col.astype(jnp.float32) + ermax
    m = jnp.maximum(mrow, SLOPE * mrow) + 0.01
    m16 = m.astype(BF)
    s = el_col + er_row
    l = jnp.maximum(s, BF(SLOPE) * s)
    pm = jnp.exp2(l - m16) * adj
    od = jnp.dot(pm, vaug, preferred_element_type=jnp.float32)
    o, denom = od[:, :-1], od[:, -1:]
    safe = denom > 0.0
    r = 1.0 / jnp.where(safe, denom, 1.0)
    return jnp.where(safe, o * r, vmean)


def _fused_kernel(x_ref, w1_ref, al_ref, ar_ref, adj_ref, w2_ref, a2l_ref,
                  a2r_ref, out_ref, g16_s, el1_s, ert_s, g2_s, g2f_s, el2_s,
                  er2_s, er2t_s):
    i = pl.program_id(0)

    @pl.when(i == 0)
    def _prologue():
        g = jnp.dot(x_ref[...], w1_ref[...], preferred_element_type=jnp.float32)
        ones = jnp.ones((N, 1), dtype=jnp.float32)
        g16_s[...] = jnp.concatenate(
            [jnp.concatenate([g[:, h * D1:(h + 1) * D1], ones], axis=1)
             for h in range(H1)], axis=1).astype(BF)
        el1_s[...] = (jnp.dot(g, al_ref[...],
                              preferred_element_type=jnp.float32) * LOG2E).astype(BF)
        er = jnp.dot(g, ar_ref[...], preferred_element_type=jnp.float32) * LOG2E
        ert_s[...] = er.T.astype(BF)

    @pl.when(i < NBLK)
    def _layer1():
        row0 = i * BI
        adj = adj_ref[...]
        gaug = g16_s[...]
        el = el1_s[pl.ds(row0, BI), :]
        outs = []
        for h in range(H1):
            vaug = gaug[:, h * (D1 + 1):(h + 1) * (D1 + 1)]
            vmean = jnp.sum(vaug[:, :-1].astype(jnp.float32),
                            axis=0, keepdims=True) * (1.0 / N)
            outs.append(_attend(el[:, h:h + 1], ert_s[h:h + 1, :], adj, vaug, vmean))
        h1 = jnp.concatenate(outs, axis=1)
        h1 = jnp.where(h1 > 0, h1, jnp.exp(jnp.minimum(h1, 0.0)) - 1.0)
        g2 = jnp.dot(h1.astype(BF), w2_ref[...], preferred_element_type=jnp.float32)
        g2f_s[pl.ds(row0, BI), :] = g2
        el2_s[pl.ds(row0, BI), :] = (jnp.dot(
            g2, a2l_ref[...], preferred_element_type=jnp.float32) * LOG2E).astype(BF)
        er2_s[pl.ds(row0, BI), :] = jnp.dot(
            g2, a2r_ref[...], preferred_element_type=jnp.float32) * LOG2E

    @pl.when(i == NBLK)
    def _finish_layer1():
        er2t_s[...] = er2_s[...].T.astype(BF)
        g2_s[...] = jnp.concatenate(
            [g2f_s[...], jnp.ones((N, 1), dtype=jnp.float32)], axis=1).astype(BF)

    @pl.when(i >= NBLK)
    def _layer2():
        row0 = (i - NBLK) * BI
        vaug = g2_s[...]
        vmean = jnp.sum(vaug[:, :-1].astype(jnp.float32),
                        axis=0, keepdims=True) * (1.0 / N)
        out_ref[...] = _attend(el2_s[pl.ds(row0, BI), :], er2t_s[...],
                               adj_ref[...], vaug, vmean)


def kernel(x, adj_mat, W1, a1_l, a1_r, W2, a2_l, a2_r):
    adj = adj_mat.reshape(N, N).astype(BF)
    eye = jnp.eye(H1, dtype=jnp.float32)
    Al = (eye[:, None, :] * a1_l[None, :, None]).reshape(F_HID, H1)
    Ar = (eye[:, None, :] * a1_r[None, :, None]).reshape(F_HID, H1)
    a2l = a2_l.reshape(D2, 1)
    a2r = a2_r.reshape(D2, 1)
    W2b = W2.astype(BF)

    out = pl.pallas_call(
        _fused_kernel,
        grid=(2 * NBLK,),
        in_specs=[
            pl.BlockSpec((N, F_IN), lambda i: (0, 0)),
            pl.BlockSpec((F_IN, F_HID), lambda i: (0, 0)),
            pl.BlockSpec((F_HID, H1), lambda i: (0, 0)),
            pl.BlockSpec((F_HID, H1), lambda i: (0, 0)),
            pl.BlockSpec((BI, N), lambda i: (jax.lax.rem(i, NBLK), 0)),
            pl.BlockSpec((F_HID, D2), lambda i: (0, 0)),
            pl.BlockSpec((D2, 1), lambda i: (0, 0)),
            pl.BlockSpec((D2, 1), lambda i: (0, 0)),
        ],
        out_specs=pl.BlockSpec((BI, D2), lambda i: (jnp.maximum(i - NBLK, 0), 0)),
        out_shape=jax.ShapeDtypeStruct((N, D2), jnp.float32),
        scratch_shapes=[
            pltpu.VMEM((N, H1 * (D1 + 1)), BF),
            pltpu.VMEM((N, H1), BF),
            pltpu.VMEM((H1, N), BF),
            pltpu.VMEM((N, D2 + 1), BF),
            pltpu.VMEM((N, D2), jnp.float32),
            pltpu.VMEM((N, 1), BF),
            pltpu.VMEM((N, 1), jnp.float32),
            pltpu.VMEM((1, N), BF),
        ],
    )(x, W1, Al, Ar, adj, W2b, a2l, a2r)
    return out
